# Initial kernel scaffold; baseline (speedup 1.0000x reference)
import jax
import jax.numpy as jnp
from jax.experimental import pallas as pl


def kernel(picture, tokens):
    raise NotImplementedError("write your pallas kernel here")


# speedup vs baseline: 7.5074x; 7.5074x over previous
import functools

import jax
import jax.numpy as jnp
from jax import lax
from jax.experimental import pallas as pl
from jax.experimental.pallas import tpu as pltpu
from jax.experimental.pallas import tpu_sc as plsc

_SEG = (0, 12, 25, 38, 51)
_B = 375
_L = 16
_NG = 24
_BPAD = _NG * _L
_ROWS = 5 * 64
_TOFF = 8


def _body(pic_hbm, tok_hbm, out_hbm, pic_v, tok_v, out_v):
    wid = lax.axis_index("s") * 2 + lax.axis_index("c")

    @pl.when(wid < _NG)
    def _work():
        pltpu.sync_copy(pic_hbm.at[wid], pic_v)
        pltpu.sync_copy(tok_hbm, tok_v)

        def _bcast(idx):
            return plsc.load_gather(
                tok_v, [jnp.full((_L,), _TOFF + idx, jnp.int32)])

        pic2 = []
        for j in range(5):
            acc = jnp.zeros((_L,), jnp.float32)
            for d in range(64):
                p = pic_v[j * 64 + d]
                acc = acc + p * p
            pic2.append(acc)

        dist = [[None] * 5 for _ in range(5)]
        for k in range(5):
            g2 = [jnp.zeros((_L,), jnp.float32) for _ in range(5)]
            sd = [jnp.zeros((_L,), jnp.float32) for _ in range(5)]
            for m in range(13):
                tkm = _bcast(k * 13 + m)
                for j in range(5):
                    p = pic_v[j * 64 + _SEG[k] + m]
                    g2[j] = g2[j] + p * p
                    df = p - tkm
                    sd[j] = sd[j] + df * df
            for j in range(5):
                dist[j][k] = (pic2[j] - g2[j]) + sd[j]

        inf = jnp.full((_L,), jnp.inf, jnp.float32)
        assign = [jnp.zeros((_L,), jnp.int32) for _ in range(5)]
        for r in range(5):
            minval = inf
            minc = jnp.zeros((_L,), jnp.int32)
            mint = jnp.zeros((_L,), jnp.int32)
            for j in range(5):
                for k in range(5):
                    upd = dist[j][k] < minval
                    minval = jnp.where(upd, dist[j][k], minval)
                    minc = jnp.where(upd, j, minc)
                    mint = jnp.where(upd, k, mint)
            for c in range(5):
                assign[c] = jnp.where(minc == c, mint, assign[c])
            if r < 4:
                for j in range(5):
                    for k in range(5):
                        dead = (minc == j) | (mint == k)
                        dist[j][k] = jnp.where(dead, inf, dist[j][k])

        zero = jnp.zeros((_L,), jnp.float32)

        def _zrow(i, carry):
            out_v[i] = zero
            return carry

        lax.fori_loop(0, _ROWS, _zrow, 0)
        for c in range(5):
            a13 = assign[c] * 13
            for m in range(13):
                val = plsc.load_gather(tok_v, [a13 + (_TOFF + m)])
                out_v[c * 64 + _SEG[c] + m] = val

        pltpu.sync_copy(out_v, out_hbm.at[wid])


@jax.jit
def kernel(picture, tokens):
    mesh = plsc.VectorSubcoreMesh(
        core_axis_name="c", subcore_axis_name="s",
        num_cores=2, num_subcores=16)
    call = functools.partial(
        pl.kernel,
        out_type=jax.ShapeDtypeStruct((_NG, _ROWS, _L), jnp.float32),
        mesh=mesh,
        compiler_params=pltpu.CompilerParams(needs_layout_passes=False),
        scratch_types=[
            pltpu.VMEM((_ROWS, _L), jnp.float32),
            pltpu.VMEM((80,), jnp.float32),
            pltpu.VMEM((_ROWS, _L), jnp.float32),
        ],
    )(_body)

    pic = picture.reshape(_B, _ROWS)
    pic = jnp.pad(pic, ((0, _BPAD - _B), (0, 0)))
    pic_g = pic.reshape(_NG, _L, _ROWS).transpose(0, 2, 1)
    tok = jnp.pad(tokens.reshape(65), (_TOFF, 15 - _TOFF))
    out_g = call(pic_g, tok)
    out = out_g.transpose(0, 2, 1).reshape(_BPAD, _ROWS)
    return out[:_B].reshape(_B, 5, 64)


# speedup vs baseline: 7.8156x; 1.0411x over previous
import functools

import jax
import jax.numpy as jnp
from jax import lax
from jax.experimental import pallas as pl
from jax.experimental.pallas import tpu as pltpu
from jax.experimental.pallas import tpu_sc as plsc

_SEG = (0, 12, 25, 38, 51)
_B = 375
_L = 16
_NG = 24
_REM = _B - (_NG - 1) * _L
_ROWS = 5 * 64
_TOFF = 8


def _body(pic_hbm, tok_hbm, out_hbm, pic_v, tok_v, out_v):
    wid = lax.axis_index("s") * 2 + lax.axis_index("c")
    lane_base = lax.iota(jnp.int32, _L) * _ROWS

    @pl.when(wid < _NG - 1)
    def _copy_full():
        pltpu.sync_copy(pic_hbm.at[pl.ds(wid * (_L * _ROWS), _L * _ROWS)],
                        pic_v)

    @pl.when(wid == _NG - 1)
    def _copy_rem():
        pltpu.sync_copy(
            pic_hbm.at[pl.ds((_NG - 1) * (_L * _ROWS), _REM * _ROWS)],
            pic_v.at[pl.ds(0, _REM * _ROWS)])

    @pl.when(wid < _NG)
    def _work():
        pltpu.sync_copy(tok_hbm, tok_v)

        def _load(word):
            return plsc.load_gather(pic_v, [lane_base + word])

        def _bcast(idx):
            return plsc.load_gather(
                tok_v, [jnp.full((_L,), _TOFF + idx, jnp.int32)])

        pic2 = []
        for j in range(5):
            acc = jnp.zeros((_L,), jnp.float32)
            for d in range(64):
                p = _load(j * 64 + d)
                acc = acc + p * p
            pic2.append(acc)

        dist = [[None] * 5 for _ in range(5)]
        for k in range(5):
            g2 = [jnp.zeros((_L,), jnp.float32) for _ in range(5)]
            sd = [jnp.zeros((_L,), jnp.float32) for _ in range(5)]
            for m in range(13):
                tkm = _bcast(k * 13 + m)
                for j in range(5):
                    p = _load(j * 64 + _SEG[k] + m)
                    g2[j] = g2[j] + p * p
                    df = p - tkm
                    sd[j] = sd[j] + df * df
            for j in range(5):
                dist[j][k] = (pic2[j] - g2[j]) + sd[j]

        inf = jnp.full((_L,), jnp.inf, jnp.float32)
        flat_ids = [jnp.full((_L,), j * 5 + k, jnp.int32)
                    for j in range(5) for k in range(5)]
        assign = [jnp.zeros((_L,), jnp.int32) for _ in range(5)]
        for r in range(5):
            items = [(dist[j][k], flat_ids[j * 5 + k])
                     for j in range(5) for k in range(5)]
            while len(items) > 1:
                nxt = []
                for i in range(0, len(items) - 1, 2):
                    (av, ai), (bv, bi) = items[i], items[i + 1]
                    upd = bv < av
                    nxt.append((jnp.where(upd, bv, av),
                                jnp.where(upd, bi, ai)))
                if len(items) % 2:
                    nxt.append(items[-1])
                items = nxt
            flat = items[0][1]
            minc = flat // 5
            mint = flat - 5 * minc
            for c in range(5):
                assign[c] = jnp.where(minc == c, mint, assign[c])
            if r < 4:
                for j in range(5):
                    for k in range(5):
                        dead = (minc == j) | (mint == k)
                        dist[j][k] = jnp.where(dead, inf, dist[j][k])

        zero = jnp.zeros((_L,), jnp.float32)

        def _store(word, val):
            plsc.store_scatter(out_v, [lane_base + word], val)

        token_words = {}
        for c in range(5):
            a13 = assign[c] * 13
            for m in range(13):
                token_words[c * 64 + _SEG[c] + m] = a13 + (_TOFF + m)
        for word in range(_ROWS):
            idx = token_words.get(word)
            if idx is None:
                _store(word, zero)
            else:
                _store(word, plsc.load_gather(tok_v, [idx]))

    @pl.when(wid < _NG - 1)
    def _out_full():
        pltpu.sync_copy(out_v,
                        out_hbm.at[pl.ds(wid * (_L * _ROWS), _L * _ROWS)])

    @pl.when(wid == _NG - 1)
    def _out_rem():
        pltpu.sync_copy(
            out_v.at[pl.ds(0, _REM * _ROWS)],
            out_hbm.at[pl.ds((_NG - 1) * (_L * _ROWS), _REM * _ROWS)])


@jax.jit
def kernel(picture, tokens):
    mesh = plsc.VectorSubcoreMesh(
        core_axis_name="c", subcore_axis_name="s",
        num_cores=2, num_subcores=16)
    call = functools.partial(
        pl.kernel,
        out_type=jax.ShapeDtypeStruct((_B * _ROWS,), jnp.float32),
        mesh=mesh,
        compiler_params=pltpu.CompilerParams(needs_layout_passes=False),
        scratch_types=[
            pltpu.VMEM((_L * _ROWS,), jnp.float32),
            pltpu.VMEM((80,), jnp.float32),
            pltpu.VMEM((_L * _ROWS,), jnp.float32),
        ],
    )(_body)

    pic = picture.reshape(_B * _ROWS)
    tok = jnp.pad(tokens.reshape(65), (_TOFF, 15 - _TOFF))
    return call(pic, tok).reshape(_B, 5, 64)


# speedup vs baseline: 8.0806x; 1.0339x over previous
import functools

import jax
import jax.numpy as jnp
from jax import lax
from jax.experimental import pallas as pl
from jax.experimental.pallas import tpu as pltpu
from jax.experimental.pallas import tpu_sc as plsc

_SEG = (0, 12, 25, 38, 51)
_B = 375
_L = 16
_NG = 24
_REM = _B - (_NG - 1) * _L
_ROWS = 5 * 64
_TS = _ROWS // _L
_STRIDE = _L + 1
_TOFF = 8


def _body(pic_hbm, tok_hbm, out_hbm, pic_raw, pic_t, tok_v, asg_v, out_raw):
    wid = lax.axis_index("s") * 2 + lax.axis_index("c")
    lanes = lax.iota(jnp.int32, _L)
    lanes17 = lanes * _STRIDE

    @pl.when(wid < _NG - 1)
    def _copy_full():
        pltpu.sync_copy(pic_hbm.at[pl.ds(wid * _L, _L)], pic_raw)

    @pl.when(wid == _NG - 1)
    def _copy_rem():
        pltpu.sync_copy(pic_hbm.at[pl.ds((_NG - 1) * _L, _REM)],
                        pic_raw.at[pl.ds(0, _REM)])

    @pl.when(wid < _NG)
    def _work():
        pltpu.sync_copy(tok_hbm, tok_v)

        for l in range(_L):
            for wb in range(_TS):
                v = pic_raw[l, pl.ds(wb * _L, _L)]
                plsc.store_scatter(
                    pic_t, [lanes17 + (wb * _L * _STRIDE + l)], v)

        def _load(word):
            return plsc.load_gather(pic_t, [lanes + word * _STRIDE])

        def _bcast(idx):
            return plsc.load_gather(
                tok_v, [jnp.full((_L,), _TOFF + idx, jnp.int32)])

        pic2 = []
        for j in range(5):
            acc = jnp.zeros((_L,), jnp.float32)
            for d in range(64):
                p = _load(j * 64 + d)
                acc = acc + p * p
            pic2.append(acc)

        dist = [[None] * 5 for _ in range(5)]
        for k in range(5):
            g2 = [jnp.zeros((_L,), jnp.float32) for _ in range(5)]
            sd = [jnp.zeros((_L,), jnp.float32) for _ in range(5)]
            for m in range(13):
                tkm = _bcast(k * 13 + m)
                for j in range(5):
                    p = _load(j * 64 + _SEG[k] + m)
                    g2[j] = g2[j] + p * p
                    df = p - tkm
                    sd[j] = sd[j] + df * df
            for j in range(5):
                dist[j][k] = (pic2[j] - g2[j]) + sd[j]

        inf = jnp.full((_L,), jnp.inf, jnp.float32)
        flat_ids = [jnp.full((_L,), j * 5 + k, jnp.int32)
                    for j in range(5) for k in range(5)]
        assign = [jnp.zeros((_L,), jnp.int32) for _ in range(5)]
        for r in range(5):
            items = [(dist[j][k], flat_ids[j * 5 + k])
                     for j in range(5) for k in range(5)]
            while len(items) > 1:
                nxt = []
                for i in range(0, len(items) - 1, 2):
                    (av, ai), (bv, bi) = items[i], items[i + 1]
                    upd = bv < av
                    nxt.append((jnp.where(upd, bv, av),
                                jnp.where(upd, bi, ai)))
                if len(items) % 2:
                    nxt.append(items[-1])
                items = nxt
            flat = items[0][1]
            minc = flat // 5
            mint = flat - 5 * minc
            for c in range(5):
                assign[c] = jnp.where(minc == c, mint, assign[c])
            if r < 4:
                for j in range(5):
                    for k in range(5):
                        dead = (minc == j) | (mint == k)
                        dist[j][k] = jnp.where(dead, inf, dist[j][k])

        for c in range(5):
            plsc.store_scatter(asg_v, [lanes + (_TOFF + c * _L)], assign[c])

        zero = jnp.zeros((_L,), jnp.float32)
        for l in range(_L):
            for wb in range(_TS):
                out_raw[l, pl.ds(wb * _L, _L)] = zero
        seg_mask = lanes < 13
        for l in range(_L):
            for c in range(5):
                a = plsc.load_gather(
                    asg_v, [jnp.full((_L,), _TOFF + c * _L + l, jnp.int32)])
                tok16 = plsc.load_gather(
                    tok_v, [a * 13 + (_TOFF + lanes)])
                plsc.store_scatter(
                    out_raw,
                    [jnp.full((_L,), l, jnp.int32),
                     lanes + (c * 64 + _SEG[c])],
                    tok16, mask=seg_mask)

    @pl.when(wid < _NG - 1)
    def _out_full():
        pltpu.sync_copy(out_raw, out_hbm.at[pl.ds(wid * _L, _L)])

    @pl.when(wid == _NG - 1)
    def _out_rem():
        pltpu.sync_copy(out_raw.at[pl.ds(0, _REM)],
                        out_hbm.at[pl.ds((_NG - 1) * _L, _REM)])


@jax.jit
def kernel(picture, tokens):
    mesh = plsc.VectorSubcoreMesh(
        core_axis_name="c", subcore_axis_name="s",
        num_cores=2, num_subcores=16)
    call = functools.partial(
        pl.kernel,
        out_type=jax.ShapeDtypeStruct((_B, _ROWS), jnp.float32),
        mesh=mesh,
        compiler_params=pltpu.CompilerParams(needs_layout_passes=False),
        scratch_types=[
            pltpu.VMEM((_L, _ROWS), jnp.float32),
            pltpu.VMEM((_ROWS * _STRIDE,), jnp.float32),
            pltpu.VMEM((80,), jnp.float32),
            pltpu.VMEM((_TOFF + 5 * _L,), jnp.int32),
            pltpu.VMEM((_L, _ROWS), jnp.float32),
        ],
    )(_body)

    pic = picture.reshape(_B, _ROWS)
    tok = jnp.pad(tokens.reshape(65), (_TOFF, 15 - _TOFF))
    return call(pic, tok).reshape(_B, 5, 64)


# speedup vs baseline: 8.3229x; 1.0300x over previous
import functools

import jax
import jax.numpy as jnp
from jax import lax
from jax.experimental import pallas as pl
from jax.experimental.pallas import tpu as pltpu
from jax.experimental.pallas import tpu_sc as plsc

_SEG = (0, 12, 25, 38, 51)
_B = 375
_L = 16
_NG = 24
_REM = _B - (_NG - 1) * _L
_ROWS = 5 * 64
_TS = _ROWS // _L
_STRIDE = _L + 1
_TOFF = 8


def _body(pic_hbm, tok_hbm, out_hbm, pic_sem, tok_sem,
          pic_raw, pic_t, tok_v, asg_v, out_raw):
    wid = lax.axis_index("s") * 2 + lax.axis_index("c")
    lanes = lax.iota(jnp.int32, _L)
    lanes17 = lanes * _STRIDE
    @pl.when(wid < _NG)
    def _work():
        tok_cp = pltpu.async_copy(tok_hbm, tok_v, tok_sem)

        @pl.when(wid < _NG - 1)
        def _copy_full():
            pltpu.sync_copy(pic_hbm.at[pl.ds(wid * _L, _L)], pic_raw)

        @pl.when(wid == _NG - 1)
        def _copy_rem():
            pltpu.sync_copy(pic_hbm.at[pl.ds((_NG - 1) * _L, _REM)],
                            pic_raw.at[pl.ds(0, _REM)])

        tok_cp.wait()

        for l in range(_L):
            for wb in range(_TS):
                v = pic_raw[l, pl.ds(wb * _L, _L)]
                plsc.store_scatter(
                    pic_t, [lanes17 + (wb * _L * _STRIDE + l)], v)

        def _load(word):
            return plsc.load_gather(pic_t, [lanes + word * _STRIDE])

        def _bcast(idx):
            return plsc.load_gather(
                tok_v, [jnp.full((_L,), _TOFF + idx, jnp.int32)])

        pic2 = []
        for j in range(5):
            acc = jnp.zeros((_L,), jnp.float32)
            for d in range(64):
                p = _load(j * 64 + d)
                acc = acc + p * p
            pic2.append(acc)

        dist = [[None] * 5 for _ in range(5)]
        for k in range(5):
            g2 = [jnp.zeros((_L,), jnp.float32) for _ in range(5)]
            sd = [jnp.zeros((_L,), jnp.float32) for _ in range(5)]
            for m in range(13):
                tkm = _bcast(k * 13 + m)
                for j in range(5):
                    p = _load(j * 64 + _SEG[k] + m)
                    g2[j] = g2[j] + p * p
                    df = p - tkm
                    sd[j] = sd[j] + df * df
            for j in range(5):
                dist[j][k] = (pic2[j] - g2[j]) + sd[j]

        inf = jnp.full((_L,), jnp.inf, jnp.float32)
        flat_ids = [jnp.full((_L,), j * 5 + k, jnp.int32)
                    for j in range(5) for k in range(5)]
        assign = [jnp.zeros((_L,), jnp.int32) for _ in range(5)]
        for r in range(5):
            items = [(dist[j][k], flat_ids[j * 5 + k])
                     for j in range(5) for k in range(5)]
            while len(items) > 1:
                nxt = []
                for i in range(0, len(items) - 1, 2):
                    (av, ai), (bv, bi) = items[i], items[i + 1]
                    upd = bv < av
                    nxt.append((jnp.where(upd, bv, av),
                                jnp.where(upd, bi, ai)))
                if len(items) % 2:
                    nxt.append(items[-1])
                items = nxt
            flat = items[0][1]
            minc = flat // 5
            mint = flat - 5 * minc
            for c in range(5):
                assign[c] = jnp.where(minc == c, mint, assign[c])
            if r < 4:
                for j in range(5):
                    for k in range(5):
                        dead = (minc == j) | (mint == k)
                        dist[j][k] = jnp.where(dead, inf, dist[j][k])

        for c in range(5):
            plsc.store_scatter(asg_v, [lanes + (_TOFF + c * _L)], assign[c])

        zero = jnp.zeros((_L,), jnp.float32)
        for l in range(_L):
            for wb in range(_TS):
                out_raw[l, pl.ds(wb * _L, _L)] = zero
        seg_mask = lanes < 13
        for l in range(_L):
            for c in range(5):
                a = plsc.load_gather(
                    asg_v, [jnp.full((_L,), _TOFF + c * _L + l, jnp.int32)])
                tok16 = plsc.load_gather(
                    tok_v, [a * 13 + (_TOFF + lanes)])
                plsc.store_scatter(
                    out_raw,
                    [jnp.full((_L,), l, jnp.int32),
                     lanes + (c * 64 + _SEG[c])],
                    tok16, mask=seg_mask)

        @pl.when(wid < _NG - 1)
        def _out_full():
            pltpu.sync_copy(out_raw, out_hbm.at[pl.ds(wid * _L, _L)])

        @pl.when(wid == _NG - 1)
        def _out_rem():
            pltpu.sync_copy(out_raw.at[pl.ds(0, _REM)],
                            out_hbm.at[pl.ds((_NG - 1) * _L, _REM)])


@jax.jit
def kernel(picture, tokens):
    mesh = plsc.VectorSubcoreMesh(
        core_axis_name="c", subcore_axis_name="s",
        num_cores=2, num_subcores=16)
    call = functools.partial(
        pl.kernel,
        out_type=jax.ShapeDtypeStruct((_B, _ROWS), jnp.float32),
        mesh=mesh,
        compiler_params=pltpu.CompilerParams(needs_layout_passes=False),
        scratch_types=[
            pltpu.SemaphoreType.DMA,
            pltpu.SemaphoreType.DMA,
            pltpu.VMEM((_L, _ROWS), jnp.float32),
            pltpu.VMEM((_ROWS * _STRIDE,), jnp.float32),
            pltpu.VMEM((80,), jnp.float32),
            pltpu.VMEM((_TOFF + 5 * _L,), jnp.int32),
            pltpu.VMEM((_L, _ROWS), jnp.float32),
        ],
    )(_body)

    pic = picture.reshape(_B, _ROWS)
    tok = jnp.pad(tokens.reshape(65), (_TOFF, 15 - _TOFF))
    return call(pic, tok).reshape(_B, 5, 64)


# speedup vs baseline: 8.8833x; 1.0673x over previous
import functools

import jax
import jax.numpy as jnp
from jax import lax
from jax.experimental import pallas as pl
from jax.experimental.pallas import tpu as pltpu
from jax.experimental.pallas import tpu_sc as plsc

_SEG = (0, 12, 25, 38, 51)
_B = 375
_L = 16
_NG = 24
_REM = _B - (_NG - 1) * _L
_ROWS = 5 * 64
_TS = _ROWS // _L
_STRIDE = _L + 1
_TOFF = 8


def _body(pic_hbm, tok_hbm, out_hbm, pic_sem, tok_sem,
          pic_raw, pic_t, tok_v, asg_v, out_raw):
    wid = lax.axis_index("s") * 2 + lax.axis_index("c")
    lanes = lax.iota(jnp.int32, _L)
    lanes17 = lanes * _STRIDE
    @pl.when(wid < _NG)
    def _work():
        tok_cp = pltpu.async_copy(tok_hbm, tok_v, tok_sem)

        @pl.when(wid < _NG - 1)
        def _copy_full():
            pltpu.sync_copy(pic_hbm.at[pl.ds(wid * _L, _L)], pic_raw)

        @pl.when(wid == _NG - 1)
        def _copy_rem():
            pltpu.sync_copy(pic_hbm.at[pl.ds((_NG - 1) * _L, _REM)],
                            pic_raw.at[pl.ds(0, _REM)])

        tok_cp.wait()

        def _restage(l, carry):
            for wb in range(_TS):
                v = pic_raw[l, pl.ds(wb * _L, _L)]
                plsc.store_scatter(
                    pic_t, [(lanes17 + wb * (_L * _STRIDE)) + l], v)
            return carry

        lax.fori_loop(0, _L, _restage, 0)

        def _load(word):
            return plsc.load_gather(pic_t, [lanes + word * _STRIDE])

        def _bcast(idx):
            return plsc.load_gather(
                tok_v, [jnp.full((_L,), _TOFF + idx, jnp.int32)])

        pic2 = []
        for j in range(5):
            acc = jnp.zeros((_L,), jnp.float32)
            for d in range(64):
                p = _load(j * 64 + d)
                acc = acc + p * p
            pic2.append(acc)

        dist = [[None] * 5 for _ in range(5)]
        for k in range(5):
            g2 = [jnp.zeros((_L,), jnp.float32) for _ in range(5)]
            sd = [jnp.zeros((_L,), jnp.float32) for _ in range(5)]
            for m in range(13):
                tkm = _bcast(k * 13 + m)
                for j in range(5):
                    p = _load(j * 64 + _SEG[k] + m)
                    g2[j] = g2[j] + p * p
                    df = p - tkm
                    sd[j] = sd[j] + df * df
            for j in range(5):
                dist[j][k] = (pic2[j] - g2[j]) + sd[j]

        inf = jnp.full((_L,), jnp.inf, jnp.float32)
        flat_ids = [jnp.full((_L,), j * 5 + k, jnp.int32)
                    for j in range(5) for k in range(5)]
        assign = [jnp.zeros((_L,), jnp.int32) for _ in range(5)]
        for r in range(5):
            items = [(dist[j][k], flat_ids[j * 5 + k])
                     for j in range(5) for k in range(5)]
            while len(items) > 1:
                nxt = []
                for i in range(0, len(items) - 1, 2):
                    (av, ai), (bv, bi) = items[i], items[i + 1]
                    upd = bv < av
                    nxt.append((jnp.where(upd, bv, av),
                                jnp.where(upd, bi, ai)))
                if len(items) % 2:
                    nxt.append(items[-1])
                items = nxt
            flat = items[0][1]
            minc = flat // 5
            mint = flat - 5 * minc
            for c in range(5):
                assign[c] = jnp.where(minc == c, mint, assign[c])
            if r < 4:
                for j in range(5):
                    for k in range(5):
                        dead = (minc == j) | (mint == k)
                        dist[j][k] = jnp.where(dead, inf, dist[j][k])

        for c in range(5):
            plsc.store_scatter(asg_v, [lanes + (_TOFF + c * _L)], assign[c])

        zero = jnp.zeros((_L,), jnp.float32)
        seg_mask = lanes < 13

        def _emit(l, carry):
            for wb in range(_TS):
                out_raw[l, pl.ds(wb * _L, _L)] = zero
            for c in range(5):
                a = plsc.load_gather(
                    asg_v, [jnp.full((_L,), _TOFF + c * _L, jnp.int32) + l])
                tok16 = plsc.load_gather(
                    tok_v, [a * 13 + (_TOFF + lanes)])
                plsc.store_scatter(
                    out_raw,
                    [jnp.full((_L,), 0, jnp.int32) + l,
                     lanes + (c * 64 + _SEG[c])],
                    tok16, mask=seg_mask)
            return carry

        lax.fori_loop(0, _L, _emit, 0)

        @pl.when(wid < _NG - 1)
        def _out_full():
            pltpu.sync_copy(out_raw, out_hbm.at[pl.ds(wid * _L, _L)])

        @pl.when(wid == _NG - 1)
        def _out_rem():
            pltpu.sync_copy(out_raw.at[pl.ds(0, _REM)],
                            out_hbm.at[pl.ds((_NG - 1) * _L, _REM)])


@jax.jit
def kernel(picture, tokens):
    mesh = plsc.VectorSubcoreMesh(
        core_axis_name="c", subcore_axis_name="s",
        num_cores=2, num_subcores=16)
    call = functools.partial(
        pl.kernel,
        out_type=jax.ShapeDtypeStruct((_B, _ROWS), jnp.float32),
        mesh=mesh,
        compiler_params=pltpu.CompilerParams(needs_layout_passes=False),
        scratch_types=[
            pltpu.SemaphoreType.DMA,
            pltpu.SemaphoreType.DMA,
            pltpu.VMEM((_L, _ROWS), jnp.float32),
            pltpu.VMEM((_ROWS * _STRIDE,), jnp.float32),
            pltpu.VMEM((80,), jnp.float32),
            pltpu.VMEM((_TOFF + 5 * _L,), jnp.int32),
            pltpu.VMEM((_L, _ROWS), jnp.float32),
        ],
    )(_body)

    pic = picture.reshape(_B, _ROWS)
    tok = jnp.pad(tokens.reshape(65), (_TOFF, 15 - _TOFF))
    return call(pic, tok).reshape(_B, 5, 64)


# speedup vs baseline: 9.9071x; 1.1153x over previous
import functools

import jax
import jax.numpy as jnp
from jax import lax
from jax.experimental import pallas as pl
from jax.experimental.pallas import tpu as pltpu
from jax.experimental.pallas import tpu_sc as plsc

_SEG = (0, 12, 25, 38, 51)
_B = 375
_L = 16
_NG = 24
_REM = _B - (_NG - 1) * _L
_ROWS = 5 * 64
_TS = _ROWS // _L
_STRIDE = _L + 1
_TOFF = 8


def _body(pic_hbm, tok_hbm, out_hbm, pic_sem, tok_sem,
          pic_raw, pic_t, tok_v, asg_v, out_raw):
    wid = lax.axis_index("s") * 2 + lax.axis_index("c")
    lanes = lax.iota(jnp.int32, _L)
    lanes17 = lanes * _STRIDE
    @pl.when(wid < _NG)
    def _work():
        tok_cp = pltpu.async_copy(tok_hbm, tok_v, tok_sem)

        @pl.when(wid < _NG - 1)
        def _copy_full():
            pltpu.sync_copy(pic_hbm.at[pl.ds(wid * _L, _L)], pic_raw)

        @pl.when(wid == _NG - 1)
        def _copy_rem():
            pltpu.sync_copy(pic_hbm.at[pl.ds((_NG - 1) * _L, _REM)],
                            pic_raw.at[pl.ds(0, _REM)])

        tok_cp.wait()

        def _restage(l, carry):
            for wb in range(_TS):
                v = pic_raw[l, pl.ds(wb * _L, _L)]
                plsc.store_scatter(
                    pic_t, [(lanes17 + wb * (_L * _STRIDE)) + l], v)
            return carry

        lax.fori_loop(0, _L, _restage, 0)

        def _load(word):
            return plsc.load_gather(pic_t, [lanes + word * _STRIDE])

        def _bcast(idx):
            return plsc.load_gather(
                tok_v, [jnp.full((_L,), _TOFF + idx, jnp.int32)])

        def _pic2_step(d, accs):
            out = []
            for j in range(5):
                p = plsc.load_gather(
                    pic_t, [lanes + (j * 64 * _STRIDE) + d * _STRIDE])
                out.append(accs[j] + p * p)
            return tuple(out)

        pic2 = list(lax.fori_loop(
            0, 64, _pic2_step,
            tuple(jnp.zeros((_L,), jnp.float32) for _ in range(5))))

        dist = [[None] * 5 for _ in range(5)]
        zeros10 = tuple(jnp.zeros((_L,), jnp.float32) for _ in range(10))
        for k in range(5):
            def _dist_step(m, accs, k=k):
                tkm = plsc.load_gather(
                    tok_v,
                    [jnp.full((_L,), _TOFF + k * 13, jnp.int32) + m])
                out = []
                for j in range(5):
                    p = plsc.load_gather(
                        pic_t,
                        [(lanes + (j * 64 + _SEG[k]) * _STRIDE)
                         + m * _STRIDE])
                    df = p - tkm
                    out.append(accs[2 * j] + p * p)
                    out.append(accs[2 * j + 1] + df * df)
                return tuple(out)

            accs = lax.fori_loop(0, 13, _dist_step, zeros10)
            for j in range(5):
                dist[j][k] = (pic2[j] - accs[2 * j]) + accs[2 * j + 1]

        inf = jnp.full((_L,), jnp.inf, jnp.float32)
        flat_ids = [jnp.full((_L,), j * 5 + k, jnp.int32)
                    for j in range(5) for k in range(5)]
        assign = [jnp.zeros((_L,), jnp.int32) for _ in range(5)]
        for r in range(5):
            items = [(dist[j][k], flat_ids[j * 5 + k])
                     for j in range(5) for k in range(5)]
            while len(items) > 1:
                nxt = []
                for i in range(0, len(items) - 1, 2):
                    (av, ai), (bv, bi) = items[i], items[i + 1]
                    upd = bv < av
                    nxt.append((jnp.where(upd, bv, av),
                                jnp.where(upd, bi, ai)))
                if len(items) % 2:
                    nxt.append(items[-1])
                items = nxt
            flat = items[0][1]
            minc = flat // 5
            mint = flat - 5 * minc
            for c in range(5):
                assign[c] = jnp.where(minc == c, mint, assign[c])
            if r < 4:
                for j in range(5):
                    for k in range(5):
                        dead = (minc == j) | (mint == k)
                        dist[j][k] = jnp.where(dead, inf, dist[j][k])

        for c in range(5):
            plsc.store_scatter(asg_v, [lanes + (_TOFF + c * _L)], assign[c])

        zero = jnp.zeros((_L,), jnp.float32)
        seg_mask = lanes < 13

        def _emit(l, carry):
            for wb in range(_TS):
                out_raw[l, pl.ds(wb * _L, _L)] = zero
            for c in range(5):
                a = plsc.load_gather(
                    asg_v, [jnp.full((_L,), _TOFF + c * _L, jnp.int32) + l])
                tok16 = plsc.load_gather(
                    tok_v, [a * 13 + (_TOFF + lanes)])
                plsc.store_scatter(
                    out_raw,
                    [jnp.full((_L,), 0, jnp.int32) + l,
                     lanes + (c * 64 + _SEG[c])],
                    tok16, mask=seg_mask)
            return carry

        lax.fori_loop(0, _L, _emit, 0)

        @pl.when(wid < _NG - 1)
        def _out_full():
            pltpu.sync_copy(out_raw, out_hbm.at[pl.ds(wid * _L, _L)])

        @pl.when(wid == _NG - 1)
        def _out_rem():
            pltpu.sync_copy(out_raw.at[pl.ds(0, _REM)],
                            out_hbm.at[pl.ds((_NG - 1) * _L, _REM)])


@jax.jit
def kernel(picture, tokens):
    mesh = plsc.VectorSubcoreMesh(
        core_axis_name="c", subcore_axis_name="s",
        num_cores=2, num_subcores=16)
    call = functools.partial(
        pl.kernel,
        out_type=jax.ShapeDtypeStruct((_B, _ROWS), jnp.float32),
        mesh=mesh,
        compiler_params=pltpu.CompilerParams(needs_layout_passes=False),
        scratch_types=[
            pltpu.SemaphoreType.DMA,
            pltpu.SemaphoreType.DMA,
            pltpu.VMEM((_L, _ROWS), jnp.float32),
            pltpu.VMEM((_ROWS * _STRIDE,), jnp.float32),
            pltpu.VMEM((80,), jnp.float32),
            pltpu.VMEM((_TOFF + 5 * _L,), jnp.int32),
            pltpu.VMEM((_L, _ROWS), jnp.float32),
        ],
    )(_body)

    pic = picture.reshape(_B, _ROWS)
    tok = jnp.pad(tokens.reshape(65), (_TOFF, 15 - _TOFF))
    return call(pic, tok).reshape(_B, 5, 64)


# speedup vs baseline: 9.9959x; 1.0090x over previous
import functools

import jax
import jax.numpy as jnp
from jax import lax
from jax.experimental import pallas as pl
from jax.experimental.pallas import tpu as pltpu
from jax.experimental.pallas import tpu_sc as plsc

_SEG = (0, 12, 25, 38, 51)
_B = 375
_L = 16
_NG = 24
_REM = _B - (_NG - 1) * _L
_ROWS = 5 * 64
_TS = _ROWS // _L
_STRIDE = _L + 1
_TOFF = 8


def _body(pic_hbm, tok_hbm, out_hbm, pic_sem, tok_sem,
          pic_raw, pic_t, tok_v, asg_v, out_raw):
    wid = lax.axis_index("s") * 2 + lax.axis_index("c")
    lanes = lax.iota(jnp.int32, _L)
    lanes17 = lanes * _STRIDE
    @pl.when(wid < _NG)
    def _work():
        tok_cp = pltpu.async_copy(tok_hbm, tok_v, tok_sem)

        @pl.when(wid < _NG - 1)
        def _copy_full():
            pltpu.sync_copy(pic_hbm.at[pl.ds(wid * _L, _L)], pic_raw)

        @pl.when(wid == _NG - 1)
        def _copy_rem():
            pltpu.sync_copy(pic_hbm.at[pl.ds((_NG - 1) * _L, _REM)],
                            pic_raw.at[pl.ds(0, _REM)])

        tok_cp.wait()

        def _restage(l, carry):
            for wb in range(_TS):
                v = pic_raw[l, pl.ds(wb * _L, _L)]
                plsc.store_scatter(
                    pic_t, [(lanes17 + wb * (_L * _STRIDE)) + l], v)
            return carry

        lax.fori_loop(0, _L, _restage, 0)

        def _load(word):
            return plsc.load_gather(pic_t, [lanes + word * _STRIDE])

        def _bcast(idx):
            return plsc.load_gather(
                tok_v, [jnp.full((_L,), _TOFF + idx, jnp.int32)])

        def _pic2_step(d, accs):
            out = []
            for j in range(5):
                p = plsc.load_gather(
                    pic_t, [lanes + (j * 64 * _STRIDE) + d * _STRIDE])
                out.append(accs[j] + p * p)
            return tuple(out)

        pic2 = list(lax.fori_loop(
            0, 64, _pic2_step,
            tuple(jnp.zeros((_L,), jnp.float32) for _ in range(5))))

        dist = [[None] * 5 for _ in range(5)]
        zeros10 = tuple(jnp.zeros((_L,), jnp.float32) for _ in range(10))
        for k in range(5):
            def _dist_step(m, accs, k=k):
                tkm = plsc.load_gather(
                    tok_v,
                    [jnp.full((_L,), _TOFF + k * 13, jnp.int32) + m])
                out = []
                for j in range(5):
                    p = plsc.load_gather(
                        pic_t,
                        [(lanes + (j * 64 + _SEG[k]) * _STRIDE)
                         + m * _STRIDE])
                    df = p - tkm
                    out.append(accs[2 * j] + p * p)
                    out.append(accs[2 * j + 1] + df * df)
                return tuple(out)

            accs = lax.fori_loop(0, 13, _dist_step, zeros10)
            for j in range(5):
                dist[j][k] = (pic2[j] - accs[2 * j]) + accs[2 * j + 1]

        inf = jnp.full((_L,), jnp.inf, jnp.float32)
        flat_ids = [jnp.full((_L,), j * 5 + k, jnp.int32)
                    for j in range(5) for k in range(5)]

        def _round(r, carry):
            dists = list(carry[:25])
            assign = list(carry[25:])
            items = list(zip(dists, flat_ids))
            while len(items) > 1:
                nxt = []
                for i in range(0, len(items) - 1, 2):
                    (av, ai), (bv, bi) = items[i], items[i + 1]
                    upd = bv < av
                    nxt.append((jnp.where(upd, bv, av),
                                jnp.where(upd, bi, ai)))
                if len(items) % 2:
                    nxt.append(items[-1])
                items = nxt
            flat = items[0][1]
            minc = flat // 5
            mint = flat - 5 * minc
            for c in range(5):
                assign[c] = jnp.where(minc == c, mint, assign[c])
            for j in range(5):
                for k in range(5):
                    dead = (minc == j) | (mint == k)
                    dists[j * 5 + k] = jnp.where(dead, inf, dists[j * 5 + k])
            return tuple(dists) + tuple(assign)

        carry0 = tuple(dist[j][k] for j in range(5) for k in range(5)) + \
            tuple(jnp.zeros((_L,), jnp.int32) for _ in range(5))
        assign = list(lax.fori_loop(0, 5, _round, carry0)[25:])

        for c in range(5):
            plsc.store_scatter(asg_v, [lanes + (_TOFF + c * _L)], assign[c])

        zero = jnp.zeros((_L,), jnp.float32)
        seg_mask = lanes < 13

        def _emit(l, carry):
            for wb in range(_TS):
                out_raw[l, pl.ds(wb * _L, _L)] = zero
            for c in range(5):
                a = plsc.load_gather(
                    asg_v, [jnp.full((_L,), _TOFF + c * _L, jnp.int32) + l])
                tok16 = plsc.load_gather(
                    tok_v, [a * 13 + (_TOFF + lanes)])
                plsc.store_scatter(
                    out_raw,
                    [jnp.full((_L,), 0, jnp.int32) + l,
                     lanes + (c * 64 + _SEG[c])],
                    tok16, mask=seg_mask)
            return carry

        lax.fori_loop(0, _L, _emit, 0)

        @pl.when(wid < _NG - 1)
        def _out_full():
            pltpu.sync_copy(out_raw, out_hbm.at[pl.ds(wid * _L, _L)])

        @pl.when(wid == _NG - 1)
        def _out_rem():
            pltpu.sync_copy(out_raw.at[pl.ds(0, _REM)],
                            out_hbm.at[pl.ds((_NG - 1) * _L, _REM)])


@jax.jit
def kernel(picture, tokens):
    mesh = plsc.VectorSubcoreMesh(
        core_axis_name="c", subcore_axis_name="s",
        num_cores=2, num_subcores=16)
    call = functools.partial(
        pl.kernel,
        out_type=jax.ShapeDtypeStruct((_B, _ROWS), jnp.float32),
        mesh=mesh,
        compiler_params=pltpu.CompilerParams(needs_layout_passes=False),
        scratch_types=[
            pltpu.SemaphoreType.DMA,
            pltpu.SemaphoreType.DMA,
            pltpu.VMEM((_L, _ROWS), jnp.float32),
            pltpu.VMEM((_ROWS * _STRIDE,), jnp.float32),
            pltpu.VMEM((80,), jnp.float32),
            pltpu.VMEM((_TOFF + 5 * _L,), jnp.int32),
            pltpu.VMEM((_L, _ROWS), jnp.float32),
        ],
    )(_body)

    pic = picture.reshape(_B, _ROWS)
    tok = jnp.pad(tokens.reshape(65), (_TOFF, 15 - _TOFF))
    return call(pic, tok).reshape(_B, 5, 64)
